# baseline (speedup 1.0000x reference)
import functools

import jax
import jax.numpy as jnp
from jax import lax
from jax.experimental import pallas as pl
from jax.experimental.pallas import tpu as pltpu
from jax.experimental.pallas import tpu_sc as plsc

_NC = 2
_NS = 16
_LN = 16
_NW = _NC * _NS

_CH = 80


def _sc_mesh():
    return plsc.VectorSubcoreMesh(core_axis_name="c", subcore_axis_name="s")


def _sc_counts(col, np_pad, h):
    e = col.shape[0]
    per_w = e // _NW
    nch = per_w // _CH
    rpt = np_pad // _NS

    @functools.partial(
        pl.kernel,
        out_type=jax.ShapeDtypeStruct((_NC, np_pad, h), jnp.float32),
        mesh=_sc_mesh(),
        scratch_types=[
            pltpu.VMEM((_CH,), jnp.int32),
            pltpu.VMEM((_CH, h), jnp.float32),
            pltpu.VMEM_SHARED((np_pad, h), jnp.float32),
        ],
    )
    def k(col_hbm, ones_hbm, zeros_hbm, out_hbm, idx_v, ones_v, acc_sh):
        cid = lax.axis_index("c")
        sid = lax.axis_index("s")
        wid = cid * _NS + sid
        pltpu.sync_copy(ones_hbm, ones_v)
        pltpu.sync_copy(zeros_hbm, acc_sh.at[pl.ds(sid * rpt, rpt)])
        plsc.subcore_barrier()
        base0 = wid * per_w

        def body(j, _):
            pltpu.sync_copy(col_hbm.at[pl.ds(base0 + j * _CH, _CH)], idx_v)
            pltpu.sync_copy(ones_v, acc_sh.at[idx_v], add=True)
            return 0

        lax.fori_loop(0, nch, body, 0)
        plsc.subcore_barrier()
        pltpu.sync_copy(acc_sh.at[pl.ds(sid * rpt, rpt)],
                        out_hbm.at[cid, pl.ds(sid * rpt, rpt)])

    return k(col, jnp.ones((_CH, h), jnp.float32),
             jnp.zeros((rpt, h), jnp.float32))


def _sc_gather(xa, xb, row, col):
    n, h = xa.shape
    e = row.shape[0]
    per_w = e // _NW
    nch = per_w // _CH

    @functools.partial(
        pl.kernel,
        out_type=jax.ShapeDtypeStruct((e, h), jnp.float32),
        mesh=_sc_mesh(),
        scratch_types=[
            [pltpu.VMEM((_CH,), jnp.int32)] * 2,
            [pltpu.VMEM((_CH,), jnp.int32)] * 2,
            [pltpu.VMEM((_CH, h), jnp.float32)] * 2,
            [pltpu.VMEM((_CH, h), jnp.float32)] * 2,
            [pltpu.VMEM((_CH, h), jnp.float32)] * 2,
            [pltpu.SemaphoreType.DMA] * 2,
            [pltpu.SemaphoreType.DMA] * 2,
            [pltpu.SemaphoreType.DMA] * 2,
            [pltpu.SemaphoreType.DMA] * 2,
            [pltpu.SemaphoreType.DMA] * 2,
        ],
    )
    def k(xa_hbm, xb_hbm, row_hbm, col_hbm, s_hbm,
          idx_r, idx_c, av, bv, ov, sir, sic, sga, sgb, sw):
        cid = lax.axis_index("c")
        sid = lax.axis_index("s")
        wid = cid * _NS + sid
        base0 = wid * per_w

        def issue_idx(j, p):
            b0 = base0 + j * _CH
            pltpu.async_copy(row_hbm.at[pl.ds(b0, _CH)], idx_r[p], sir[p])
            pltpu.async_copy(col_hbm.at[pl.ds(b0, _CH)], idx_c[p], sic[p])

        def wait_idx(p):
            pltpu.make_async_copy(row_hbm.at[pl.ds(0, _CH)], idx_r[p], sir[p]).wait()
            pltpu.make_async_copy(col_hbm.at[pl.ds(0, _CH)], idx_c[p], sic[p]).wait()

        def issue_gather(p):
            pltpu.async_copy(xa_hbm.at[idx_r[p]], av[p], sga[p])
            pltpu.async_copy(xb_hbm.at[idx_c[p]], bv[p], sgb[p])

        def wait_gather(p):
            pltpu.make_async_copy(xa_hbm.at[idx_r[p]], av[p], sga[p]).wait()
            pltpu.make_async_copy(xb_hbm.at[idx_c[p]], bv[p], sgb[p]).wait()

        def step(j, p):
            p2 = 1 - p

            @pl.when(j >= 2)
            def _():
                pltpu.make_async_copy(ov[p], s_hbm.at[pl.ds(0, _CH)], sw[p]).wait()

            wait_gather(p)

            @pl.when(j + 1 < nch)
            def _():
                wait_idx(p2)
                issue_gather(p2)

            @pl.when(j + 2 < nch)
            def _():
                issue_idx(j + 2, p)

            @plsc.parallel_loop(0, _CH, step=1, unroll=4)
            def _(r):
                for t in range(h // _LN):
                    sl = pl.ds(t * _LN, _LN)
                    ov[p][r, sl] = av[p][r, sl] + bv[p][r, sl]

            pltpu.async_copy(ov[p], s_hbm.at[pl.ds(base0 + j * _CH, _CH)], sw[p])

        issue_idx(0, 0)
        wait_idx(0)
        issue_gather(0)
        issue_idx(1, 1)

        def pair(g, _):
            step(2 * g, 0)
            step(2 * g + 1, 1)
            return 0

        lax.fori_loop(0, nch // 2, pair, 0)
        if nch % 2:
            step(nch - 1, 0)
        pltpu.make_async_copy(ov[0], s_hbm.at[pl.ds(0, _CH)], sw[0]).wait()
        pltpu.make_async_copy(ov[1], s_hbm.at[pl.ds(0, _CH)], sw[1]).wait()

    return k(xa, xb, row, col)


def _sc_scatter(msg, col, np_pad):
    e, h = msg.shape
    per_w = e // _NW
    nch = per_w // _CH
    rpt = np_pad // _NS

    @functools.partial(
        pl.kernel,
        out_type=jax.ShapeDtypeStruct((_NC, np_pad, h), jnp.float32),
        mesh=_sc_mesh(),
        scratch_types=[
            [pltpu.VMEM((_CH,), jnp.int32)] * 2,
            [pltpu.VMEM((_CH, h), jnp.float32)] * 2,
            [pltpu.SemaphoreType.DMA] * 2,
            [pltpu.SemaphoreType.DMA] * 2,
            [pltpu.SemaphoreType.DMA] * 2,
            pltpu.VMEM_SHARED((np_pad, h), jnp.float32),
        ],
    )
    def k(msg_hbm, col_hbm, zeros_hbm, out_hbm,
          idx_v, mv, sli, slm, ssc, acc_sh):
        cid = lax.axis_index("c")
        sid = lax.axis_index("s")
        wid = cid * _NS + sid
        pltpu.sync_copy(zeros_hbm, acc_sh.at[pl.ds(sid * rpt, rpt)])
        plsc.subcore_barrier()
        base0 = wid * per_w

        def issue_load(j, p):
            b0 = base0 + j * _CH
            pltpu.async_copy(col_hbm.at[pl.ds(b0, _CH)], idx_v[p], sli[p])
            pltpu.async_copy(msg_hbm.at[pl.ds(b0, _CH)], mv[p], slm[p])

        def wait_load(p):
            pltpu.make_async_copy(col_hbm.at[pl.ds(0, _CH)], idx_v[p], sli[p]).wait()
            pltpu.make_async_copy(msg_hbm.at[pl.ds(0, _CH)], mv[p], slm[p]).wait()

        def step(j, p):
            p2 = 1 - p
            wait_load(p)
            pltpu.async_copy(mv[p], acc_sh.at[idx_v[p]], ssc[p], add=True)

            @pl.when(j + 1 < nch)
            def _():
                @pl.when(j >= 1)
                def _():
                    pltpu.make_async_copy(
                        mv[p2], acc_sh.at[idx_v[p2]], ssc[p2]).wait()

                issue_load(j + 1, p2)

        issue_load(0, 0)

        def pair(g, _):
            step(2 * g, 0)
            step(2 * g + 1, 1)
            return 0

        lax.fori_loop(0, nch // 2, pair, 0)
        if nch % 2:
            step(nch - 1, 0)
        pltpu.make_async_copy(mv[0], acc_sh.at[idx_v[0]], ssc[0]).wait()
        pltpu.make_async_copy(mv[1], acc_sh.at[idx_v[1]], ssc[1]).wait()
        plsc.subcore_barrier()
        pltpu.sync_copy(acc_sh.at[pl.ds(sid * rpt, rpt)],
                        out_hbm.at[cid, pl.ds(sid * rpt, rpt)])

    return k(msg, col, jnp.zeros((rpt, h), jnp.float32))


def _tc_embed(nf, w_node, b_node, wa, wb, nb):
    n, d = nf.shape
    h = w_node.shape[1]

    def body(nf_ref, wn_ref, bn_ref, wa_ref, wb_ref, x_ref, xa_ref, xb_ref):
        x = jnp.dot(nf_ref[...], wn_ref[...],
                    preferred_element_type=jnp.float32) + bn_ref[...]
        x_ref[...] = x
        xa_ref[...] = jnp.dot(x, wa_ref[...], preferred_element_type=jnp.float32)
        xb_ref[...] = jnp.dot(x, wb_ref[...], preferred_element_type=jnp.float32)

    grid = n // nb
    full = lambda i: (0, 0)
    return pl.pallas_call(
        body,
        grid=grid,
        in_specs=[
            pl.BlockSpec((nb, d), lambda i: (i, 0)),
            pl.BlockSpec((d, h), full),
            pl.BlockSpec((1, h), full),
            pl.BlockSpec((h, h), full),
            pl.BlockSpec((h, h), full),
        ],
        out_specs=[
            pl.BlockSpec((nb, h), lambda i: (i, 0)),
            pl.BlockSpec((nb, h), lambda i: (i, 0)),
            pl.BlockSpec((nb, h), lambda i: (i, 0)),
        ],
        out_shape=[
            jax.ShapeDtypeStruct((n, h), jnp.float32),
            jax.ShapeDtypeStruct((n, h), jnp.float32),
            jax.ShapeDtypeStruct((n, h), jnp.float32),
        ],
    )(nf, w_node, b_node, wa, wb)


def _tc_msg(s, ef, wc, bc, w2, b2, eb):
    e, h = s.shape
    de = ef.shape[1]

    def body(s_ref, ef_ref, wc_ref, bc_ref, w2_ref, b2_ref, o_ref):
        pre = s_ref[...] + jnp.dot(ef_ref[...], wc_ref[...],
                                   preferred_element_type=jnp.float32) + bc_ref[...]
        hid = jnp.maximum(pre, 0.0)
        o_ref[...] = jnp.dot(hid, w2_ref[...],
                             preferred_element_type=jnp.float32) + b2_ref[...]

    grid = e // eb
    full = lambda i: (0, 0)
    return pl.pallas_call(
        body,
        grid=grid,
        in_specs=[
            pl.BlockSpec((eb, h), lambda i: (i, 0)),
            pl.BlockSpec((eb, de), lambda i: (i, 0)),
            pl.BlockSpec((de, h), full),
            pl.BlockSpec((1, h), full),
            pl.BlockSpec((h, h), full),
            pl.BlockSpec((1, h), full),
        ],
        out_specs=pl.BlockSpec((eb, h), lambda i: (i, 0)),
        out_shape=jax.ShapeDtypeStruct((e, h), jnp.float32),
    )(s, ef, wc, bc, w2, b2)


def _tc_update(x, part, cnt, u1a, u1b, c1, u2, c2, wa, wb, nb):
    n, h = x.shape

    def body(x_ref, p_ref, c_ref, u1a_ref, u1b_ref, c1_ref, u2_ref, c2_ref,
             wa_ref, wb_ref, xo_ref, xao_ref, xbo_ref, cs_ref):
        i = pl.program_id(0)
        cnt_tot = c_ref[0][:, 0:1] + c_ref[1][:, 0:1]
        inv = 1.0 / jnp.maximum(cnt_tot, 1.0)
        agg = (p_ref[0] + p_ref[1]) * inv
        hu = jnp.maximum(
            jnp.dot(x_ref[...], u1a_ref[...], preferred_element_type=jnp.float32)
            + jnp.dot(agg, u1b_ref[...], preferred_element_type=jnp.float32)
            + c1_ref[...], 0.0)
        xn = jnp.maximum(
            jnp.dot(hu, u2_ref[...], preferred_element_type=jnp.float32)
            + c2_ref[...], 0.0)
        xo_ref[...] = xn
        xao_ref[...] = jnp.dot(xn, wa_ref[...], preferred_element_type=jnp.float32)
        xbo_ref[...] = jnp.dot(xn, wb_ref[...], preferred_element_type=jnp.float32)

        @pl.when(i == 0)
        def _():
            cs_ref[...] = jnp.zeros_like(cs_ref)

        cs_ref[...] += jnp.broadcast_to(jnp.sum(xn, axis=0, keepdims=True), (8, h))

    grid = n // nb
    full = lambda i: (0, 0)
    return pl.pallas_call(
        body,
        grid=grid,
        in_specs=[
            pl.BlockSpec((nb, h), lambda i: (i, 0)),
            pl.BlockSpec((_NC, nb, h), lambda i: (0, i, 0)),
            pl.BlockSpec((_NC, nb, h), lambda i: (0, i, 0)),
            pl.BlockSpec((h, h), full),
            pl.BlockSpec((h, h), full),
            pl.BlockSpec((1, h), full),
            pl.BlockSpec((h, h), full),
            pl.BlockSpec((1, h), full),
            pl.BlockSpec((h, h), full),
            pl.BlockSpec((h, h), full),
        ],
        out_specs=[
            pl.BlockSpec((nb, h), lambda i: (i, 0)),
            pl.BlockSpec((nb, h), lambda i: (i, 0)),
            pl.BlockSpec((nb, h), lambda i: (i, 0)),
            pl.BlockSpec((8, h), full),
        ],
        out_shape=[
            jax.ShapeDtypeStruct((n, h), jnp.float32),
            jax.ShapeDtypeStruct((n, h), jnp.float32),
            jax.ShapeDtypeStruct((n, h), jnp.float32),
            jax.ShapeDtypeStruct((8, h), jnp.float32),
        ],
    )(x, part, cnt, u1a, u1b, c1, u2, c2, wa, wb)


def _tc_readout(colsum8, n, w1, b1, w2, b2):
    h = colsum8.shape[1]

    def body(cs_ref, w1_ref, b1_ref, w2_ref, b2_ref, o_ref):
        g = cs_ref[0:1, :] * (1.0 / n)
        hid = jnp.maximum(
            jnp.dot(g, w1_ref[...], preferred_element_type=jnp.float32)
            + b1_ref[...], 0.0)
        o_ref[...] = jnp.dot(hid, w2_ref[...],
                             preferred_element_type=jnp.float32) + b2_ref[...]

    full = lambda: (0, 0)
    return pl.pallas_call(
        body,
        grid=(),
        in_specs=[
            pl.BlockSpec((8, h), full),
            pl.BlockSpec((h, h), full),
            pl.BlockSpec((1, h), full),
            pl.BlockSpec((h, h), full),
            pl.BlockSpec((1, h), full),
        ],
        out_specs=pl.BlockSpec((1, h), full),
        out_shape=jax.ShapeDtypeStruct((1, h), jnp.float32),
    )(colsum8, w1, b1, w2, b2)


def kernel(node_features, edge_index, edge_features, params):
    n, d_node = node_features.shape
    e = edge_index.shape[1]
    h = params['node_W'].shape[1]
    row = edge_index[0]
    col = edge_index[1]
    np_pad = ((n + _NW * _LN - 1) // (_NW * _LN)) * (_NW * _LN)

    layers = params['layers']
    was, wbs, wcs, bcs = [], [], [], []
    for lp in layers:
        w1 = lp['msg_W1']
        was.append(w1[:h])
        wbs.append(w1[h:2 * h])
        wc = params['edge_W'] @ w1[2 * h:]
        bc = params['edge_b'] @ w1[2 * h:] + lp['msg_b1']
        wcs.append(wc)
        bcs.append(bc.reshape(1, h))

    nb = 2000 if (n >= 2000 and n % 2000 == 0) else n
    eb = 2000 if (e >= 2000 and e % 2000 == 0) else e

    cnt = _sc_counts(col, np_pad, h)

    x, xa, xb = _tc_embed(
        node_features, params['node_W'], params['node_b'].reshape(1, h),
        was[0], wbs[0], nb)

    colsum8 = None
    for li, lp in enumerate(layers):
        s = _sc_gather(xa, xb, row, col)
        msg = _tc_msg(s, edge_features, wcs[li], bcs[li],
                      lp['msg_W2'], lp['msg_b2'].reshape(1, h), eb)
        part = _sc_scatter(msg, col, np_pad)
        nxt = layers[li + 1] if li + 1 < len(layers) else layers[li]
        wa_n = nxt['msg_W1'][:h]
        wb_n = nxt['msg_W1'][h:2 * h]
        x, xa, xb, colsum8 = _tc_update(
            x, part, cnt,
            lp['upd_W1'][:h], lp['upd_W1'][h:], lp['upd_b1'].reshape(1, h),
            lp['upd_W2'], lp['upd_b2'].reshape(1, h),
            wa_n, wb_n, nb)

    return _tc_readout(colsum8, n, params['ro_W1'], params['ro_b1'].reshape(1, h),
                       params['ro_W2'], params['ro_b2'].reshape(1, h))


# speedup vs baseline: 1.0968x; 1.0968x over previous
import functools

import jax
import jax.numpy as jnp
from jax import lax
from jax.experimental import pallas as pl
from jax.experimental.pallas import tpu as pltpu
from jax.experimental.pallas import tpu_sc as plsc

_NC = 2
_NS = 16
_LN = 16
_NW = _NC * _NS

_CH = 80


def _sc_mesh():
    return plsc.VectorSubcoreMesh(core_axis_name="c", subcore_axis_name="s")


def _sc_counts(col, np_pad, h):
    e = col.shape[0]
    per_w = e // _NW
    nch = per_w // _CH
    rpt = np_pad // _NS

    @functools.partial(
        pl.kernel,
        out_type=jax.ShapeDtypeStruct((_NC, np_pad, h), jnp.float32),
        mesh=_sc_mesh(),
        scratch_types=[
            pltpu.VMEM((_CH,), jnp.int32),
            pltpu.VMEM((_CH, h), jnp.float32),
            pltpu.VMEM_SHARED((np_pad, h), jnp.float32),
        ],
    )
    def k(col_hbm, ones_hbm, zeros_hbm, out_hbm, idx_v, ones_v, acc_sh):
        cid = lax.axis_index("c")
        sid = lax.axis_index("s")
        wid = cid * _NS + sid
        pltpu.sync_copy(ones_hbm, ones_v)
        pltpu.sync_copy(zeros_hbm, acc_sh.at[pl.ds(sid * rpt, rpt)])
        plsc.subcore_barrier()
        base0 = wid * per_w

        def body(j, _):
            pltpu.sync_copy(col_hbm.at[pl.ds(base0 + j * _CH, _CH)], idx_v)
            pltpu.sync_copy(ones_v, acc_sh.at[idx_v], add=True)
            return 0

        lax.fori_loop(0, nch, body, 0)
        plsc.subcore_barrier()
        pltpu.sync_copy(acc_sh.at[pl.ds(sid * rpt, rpt)],
                        out_hbm.at[cid, pl.ds(sid * rpt, rpt)])

    return k(col, jnp.ones((_CH, h), jnp.float32),
             jnp.zeros((rpt, h), jnp.float32))


def _sc_gather(xa, xb, row, col):
    n, h = xa.shape
    e = row.shape[0]
    per_w = e // _NW
    nch = per_w // _CH

    @functools.partial(
        pl.kernel,
        out_type=jax.ShapeDtypeStruct((e, h), jnp.float32),
        mesh=_sc_mesh(),
        scratch_types=[
            [pltpu.VMEM((_CH,), jnp.int32)] * 2,
            [pltpu.VMEM((_CH,), jnp.int32)] * 2,
            [pltpu.VMEM((_CH, h), jnp.float32)] * 2,
            [pltpu.VMEM((_CH, h), jnp.float32)] * 2,
            [pltpu.VMEM((_CH, h), jnp.float32)] * 2,
            [pltpu.SemaphoreType.DMA] * 2,
            [pltpu.SemaphoreType.DMA] * 2,
            [pltpu.SemaphoreType.DMA] * 2,
            [pltpu.SemaphoreType.DMA] * 2,
            [pltpu.SemaphoreType.DMA] * 2,
        ],
    )
    def k(xa_hbm, xb_hbm, row_hbm, col_hbm, s_hbm,
          idx_r, idx_c, av, bv, ov, sir, sic, sga, sgb, sw):
        cid = lax.axis_index("c")
        sid = lax.axis_index("s")
        wid = cid * _NS + sid
        base0 = wid * per_w

        def issue_idx(j, p):
            b0 = base0 + j * _CH
            pltpu.async_copy(row_hbm.at[pl.ds(b0, _CH)], idx_r[p], sir[p])
            pltpu.async_copy(col_hbm.at[pl.ds(b0, _CH)], idx_c[p], sic[p])

        def wait_idx(p):
            pltpu.make_async_copy(row_hbm.at[pl.ds(0, _CH)], idx_r[p], sir[p]).wait()
            pltpu.make_async_copy(col_hbm.at[pl.ds(0, _CH)], idx_c[p], sic[p]).wait()

        def issue_gather(p):
            pltpu.async_copy(xa_hbm.at[idx_r[p]], av[p], sga[p])
            pltpu.async_copy(xb_hbm.at[idx_c[p]], bv[p], sgb[p])

        def wait_gather(p):
            pltpu.make_async_copy(xa_hbm.at[idx_r[p]], av[p], sga[p]).wait()
            pltpu.make_async_copy(xb_hbm.at[idx_c[p]], bv[p], sgb[p]).wait()

        def step(j, p):
            p2 = 1 - p

            @pl.when(j >= 2)
            def _():
                pltpu.make_async_copy(ov[p], s_hbm.at[pl.ds(0, _CH)], sw[p]).wait()

            wait_gather(p)

            @pl.when(j + 1 < nch)
            def _():
                wait_idx(p2)
                issue_gather(p2)

            @pl.when(j + 2 < nch)
            def _():
                issue_idx(j + 2, p)

            @plsc.parallel_loop(0, _CH, step=1, unroll=4)
            def _(r):
                for t in range(h // _LN):
                    sl = pl.ds(t * _LN, _LN)
                    ov[p][r, sl] = av[p][r, sl] + bv[p][r, sl]

            pltpu.async_copy(ov[p], s_hbm.at[pl.ds(base0 + j * _CH, _CH)], sw[p])

        issue_idx(0, 0)
        wait_idx(0)
        issue_gather(0)
        issue_idx(1, 1)

        def pair(g, _):
            step(2 * g, 0)
            step(2 * g + 1, 1)
            return 0

        lax.fori_loop(0, nch // 2, pair, 0)
        if nch % 2:
            step(nch - 1, 0)
        pltpu.make_async_copy(ov[0], s_hbm.at[pl.ds(0, _CH)], sw[0]).wait()
        pltpu.make_async_copy(ov[1], s_hbm.at[pl.ds(0, _CH)], sw[1]).wait()

    return k(xa, xb, row, col)


def _sc_scatter(msg, col, np_pad):
    e, h = msg.shape
    per_w = e // _NW
    nch = per_w // _CH
    rpt = np_pad // _NS

    @functools.partial(
        pl.kernel,
        out_type=jax.ShapeDtypeStruct((_NC, np_pad, h), jnp.float32),
        mesh=_sc_mesh(),
        scratch_types=[
            [pltpu.VMEM((_CH,), jnp.int32)] * 2,
            [pltpu.VMEM((_CH, h), jnp.float32)] * 2,
            [pltpu.SemaphoreType.DMA] * 2,
            [pltpu.SemaphoreType.DMA] * 2,
            [pltpu.SemaphoreType.DMA] * 2,
            pltpu.VMEM_SHARED((np_pad, h), jnp.float32),
        ],
    )
    def k(msg_hbm, col_hbm, zeros_hbm, out_hbm,
          idx_v, mv, sli, slm, ssc, acc_sh):
        cid = lax.axis_index("c")
        sid = lax.axis_index("s")
        wid = cid * _NS + sid
        pltpu.sync_copy(zeros_hbm, acc_sh.at[pl.ds(sid * rpt, rpt)])
        plsc.subcore_barrier()
        base0 = wid * per_w

        def issue_load(j, p):
            b0 = base0 + j * _CH
            pltpu.async_copy(col_hbm.at[pl.ds(b0, _CH)], idx_v[p], sli[p])
            pltpu.async_copy(msg_hbm.at[pl.ds(b0, _CH)], mv[p], slm[p])

        def wait_load(p):
            pltpu.make_async_copy(col_hbm.at[pl.ds(0, _CH)], idx_v[p], sli[p]).wait()
            pltpu.make_async_copy(msg_hbm.at[pl.ds(0, _CH)], mv[p], slm[p]).wait()

        def step(j, p):
            p2 = 1 - p
            wait_load(p)
            pltpu.async_copy(mv[p], acc_sh.at[idx_v[p]], ssc[p], add=True)

            @pl.when(j + 1 < nch)
            def _():
                @pl.when(j >= 1)
                def _():
                    pltpu.make_async_copy(
                        mv[p2], acc_sh.at[idx_v[p2]], ssc[p2]).wait()

                issue_load(j + 1, p2)

        issue_load(0, 0)

        def pair(g, _):
            step(2 * g, 0)
            step(2 * g + 1, 1)
            return 0

        lax.fori_loop(0, nch // 2, pair, 0)
        if nch % 2:
            step(nch - 1, 0)
        pltpu.make_async_copy(mv[0], acc_sh.at[idx_v[0]], ssc[0]).wait()
        pltpu.make_async_copy(mv[1], acc_sh.at[idx_v[1]], ssc[1]).wait()
        plsc.subcore_barrier()
        pltpu.sync_copy(acc_sh.at[pl.ds(sid * rpt, rpt)],
                        out_hbm.at[cid, pl.ds(sid * rpt, rpt)])

    return k(msg, col, jnp.zeros((rpt, h), jnp.float32))


def _tc_embed(nf, w_node, b_node, wa, wb, nb):
    n, d = nf.shape
    h = w_node.shape[1]

    def body(nf_ref, wn_ref, bn_ref, wa_ref, wb_ref, x_ref, xa_ref, xb_ref):
        x = jnp.dot(nf_ref[...], wn_ref[...],
                    preferred_element_type=jnp.float32) + bn_ref[...]
        x_ref[...] = x
        xa_ref[...] = jnp.dot(x, wa_ref[...], preferred_element_type=jnp.float32)
        xb_ref[...] = jnp.dot(x, wb_ref[...], preferred_element_type=jnp.float32)

    grid = n // nb
    full = lambda i: (0, 0)
    return pl.pallas_call(
        body,
        grid=grid,
        in_specs=[
            pl.BlockSpec((nb, d), lambda i: (i, 0)),
            pl.BlockSpec((d, h), full),
            pl.BlockSpec((1, h), full),
            pl.BlockSpec((h, h), full),
            pl.BlockSpec((h, h), full),
        ],
        out_specs=[
            pl.BlockSpec((nb, h), lambda i: (i, 0)),
            pl.BlockSpec((nb, h), lambda i: (i, 0)),
            pl.BlockSpec((nb, h), lambda i: (i, 0)),
        ],
        out_shape=[
            jax.ShapeDtypeStruct((n, h), jnp.float32),
            jax.ShapeDtypeStruct((n, h), jnp.float32),
            jax.ShapeDtypeStruct((n, h), jnp.float32),
        ],
    )(nf, w_node, b_node, wa, wb)


def _tc_msg(s, ef, wc, bc, w2, b2, eb):
    e, h = s.shape
    de = ef.shape[1]

    def body(s_ref, ef_ref, wc_ref, bc_ref, w2_ref, b2_ref, o_ref):
        pre = s_ref[...] + jnp.dot(ef_ref[...], wc_ref[...],
                                   preferred_element_type=jnp.float32) + bc_ref[...]
        hid = jnp.maximum(pre, 0.0)
        o_ref[...] = jnp.dot(hid, w2_ref[...],
                             preferred_element_type=jnp.float32) + b2_ref[...]

    grid = e // eb
    full = lambda i: (0, 0)
    return pl.pallas_call(
        body,
        grid=grid,
        in_specs=[
            pl.BlockSpec((eb, h), lambda i: (i, 0)),
            pl.BlockSpec((eb, de), lambda i: (i, 0)),
            pl.BlockSpec((de, h), full),
            pl.BlockSpec((1, h), full),
            pl.BlockSpec((h, h), full),
            pl.BlockSpec((1, h), full),
        ],
        out_specs=pl.BlockSpec((eb, h), lambda i: (i, 0)),
        out_shape=jax.ShapeDtypeStruct((e, h), jnp.float32),
    )(s, ef, wc, bc, w2, b2)


def _tc_update(x, part, part2, cnt, u1a, u1b, c1, u2, c2, wa, wb, nb):
    n, h = x.shape

    def body(x_ref, p_ref, q_ref, c_ref, u1a_ref, u1b_ref, c1_ref, u2_ref,
             c2_ref, wa_ref, wb_ref, xo_ref, xao_ref, xbo_ref, cs_ref):
        i = pl.program_id(0)
        cnt_tot = c_ref[0][:, 0:1] + c_ref[1][:, 0:1]
        inv = 1.0 / jnp.maximum(cnt_tot, 1.0)
        agg = (p_ref[0] + p_ref[1] + q_ref[0] + q_ref[1]) * inv
        hu = jnp.maximum(
            jnp.dot(x_ref[...], u1a_ref[...], preferred_element_type=jnp.float32)
            + jnp.dot(agg, u1b_ref[...], preferred_element_type=jnp.float32)
            + c1_ref[...], 0.0)
        xn = jnp.maximum(
            jnp.dot(hu, u2_ref[...], preferred_element_type=jnp.float32)
            + c2_ref[...], 0.0)
        xo_ref[...] = xn
        xao_ref[...] = jnp.dot(xn, wa_ref[...], preferred_element_type=jnp.float32)
        xbo_ref[...] = jnp.dot(xn, wb_ref[...], preferred_element_type=jnp.float32)

        @pl.when(i == 0)
        def _():
            cs_ref[...] = jnp.zeros_like(cs_ref)

        cs_ref[...] += jnp.broadcast_to(jnp.sum(xn, axis=0, keepdims=True), (8, h))

    grid = n // nb
    full = lambda i: (0, 0)
    return pl.pallas_call(
        body,
        grid=grid,
        in_specs=[
            pl.BlockSpec((nb, h), lambda i: (i, 0)),
            pl.BlockSpec((_NC, nb, h), lambda i: (0, i, 0)),
            pl.BlockSpec((_NC, nb, h), lambda i: (0, i, 0)),
            pl.BlockSpec((_NC, nb, h), lambda i: (0, i, 0)),
            pl.BlockSpec((h, h), full),
            pl.BlockSpec((h, h), full),
            pl.BlockSpec((1, h), full),
            pl.BlockSpec((h, h), full),
            pl.BlockSpec((1, h), full),
            pl.BlockSpec((h, h), full),
            pl.BlockSpec((h, h), full),
        ],
        out_specs=[
            pl.BlockSpec((nb, h), lambda i: (i, 0)),
            pl.BlockSpec((nb, h), lambda i: (i, 0)),
            pl.BlockSpec((nb, h), lambda i: (i, 0)),
            pl.BlockSpec((8, h), full),
        ],
        out_shape=[
            jax.ShapeDtypeStruct((n, h), jnp.float32),
            jax.ShapeDtypeStruct((n, h), jnp.float32),
            jax.ShapeDtypeStruct((n, h), jnp.float32),
            jax.ShapeDtypeStruct((8, h), jnp.float32),
        ],
    )(x, part, part2, cnt, u1a, u1b, c1, u2, c2, wa, wb)


def _tc_readout(colsum8, n, w1, b1, w2, b2):
    h = colsum8.shape[1]

    def body(cs_ref, w1_ref, b1_ref, w2_ref, b2_ref, o_ref):
        g = cs_ref[0:1, :] * (1.0 / n)
        hid = jnp.maximum(
            jnp.dot(g, w1_ref[...], preferred_element_type=jnp.float32)
            + b1_ref[...], 0.0)
        o_ref[...] = jnp.dot(hid, w2_ref[...],
                             preferred_element_type=jnp.float32) + b2_ref[...]

    full = lambda: (0, 0)
    return pl.pallas_call(
        body,
        grid=(),
        in_specs=[
            pl.BlockSpec((8, h), full),
            pl.BlockSpec((h, h), full),
            pl.BlockSpec((1, h), full),
            pl.BlockSpec((h, h), full),
            pl.BlockSpec((1, h), full),
        ],
        out_specs=pl.BlockSpec((1, h), full),
        out_shape=jax.ShapeDtypeStruct((1, h), jnp.float32),
    )(colsum8, w1, b1, w2, b2)


def kernel(node_features, edge_index, edge_features, params):
    n, d_node = node_features.shape
    e = edge_index.shape[1]
    h = params['node_W'].shape[1]
    row = edge_index[0]
    col = edge_index[1]
    np_pad = ((n + _NW * _LN - 1) // (_NW * _LN)) * (_NW * _LN)

    layers = params['layers']
    was, wbs, wcs, bcs = [], [], [], []
    for lp in layers:
        w1 = lp['msg_W1']
        was.append(w1[:h])
        wbs.append(w1[h:2 * h])
        wc = params['edge_W'] @ w1[2 * h:]
        bc = params['edge_b'] @ w1[2 * h:] + lp['msg_b1']
        wcs.append(wc)
        bcs.append(bc.reshape(1, h))

    nb = 2000 if (n >= 2000 and n % 2000 == 0) else n

    grain = _NW * _CH
    e1 = (e // (2 * grain)) * grain
    if e1 == 0 or e1 == e:
        splits = [(0, e)]
    else:
        splits = [(0, e1), (e1, e)]

    def eb_for(m):
        for cand in (2560, 2000):
            if m % cand == 0:
                return cand
        return m

    cnt = _sc_counts(col, np_pad, h)

    x, xa, xb = _tc_embed(
        node_features, params['node_W'], params['node_b'].reshape(1, h),
        was[0], wbs[0], nb)

    colsum8 = None
    for li, lp in enumerate(layers):
        parts = []
        for (lo, hi) in splits:
            s = _sc_gather(xa, xb, row[lo:hi], col[lo:hi])
            msg = _tc_msg(s, edge_features[lo:hi], wcs[li], bcs[li],
                          lp['msg_W2'], lp['msg_b2'].reshape(1, h),
                          eb_for(hi - lo))
            parts.append(_sc_scatter(msg, col[lo:hi], np_pad))
        if len(parts) == 1:
            parts = [parts[0], jnp.zeros_like(parts[0])]
        nxt = layers[li + 1] if li + 1 < len(layers) else layers[li]
        wa_n = nxt['msg_W1'][:h]
        wb_n = nxt['msg_W1'][h:2 * h]
        x, xa, xb, colsum8 = _tc_update(
            x, parts[0], parts[1], cnt,
            lp['upd_W1'][:h], lp['upd_W1'][h:], lp['upd_b1'].reshape(1, h),
            lp['upd_W2'], lp['upd_b2'].reshape(1, h),
            wa_n, wb_n, nb)

    return _tc_readout(colsum8, n, params['ro_W1'], params['ro_b1'].reshape(1, h),
                       params['ro_W2'], params['ro_b2'].reshape(1, h))
